# baseline (speedup 1.0000x reference)
import jax
import jax.numpy as jnp
from jax import lax
from jax.experimental import pallas as pl

B, C, D = 1024, 512, 64
P = C // 2
BB = 16


def _body(x_ref, masked_ref, idx_ref):
    x = x_ref[...]
    y = x * x
    lane = lax.broadcasted_iota(jnp.int32, (BB, P, 128), 2)
    y_even = jnp.where(lane < D, y, 0.0)
    s_pair = jnp.sum(y, axis=2)
    sa = jnp.sum(y_even, axis=2)
    sb = s_pair - sa

    p_iota = lax.broadcasted_iota(jnp.int32, (BB, P), 1)
    ma = jnp.max(sa, axis=1, keepdims=True)
    ia = jnp.min(jnp.where(sa >= ma, p_iota, P), axis=1, keepdims=True)
    mb = jnp.max(sb, axis=1, keepdims=True)
    ib = jnp.min(jnp.where(sb >= mb, p_iota, P), axis=1, keepdims=True)
    choose_even = (ma > mb) | ((ma == mb) & (ia <= ib))
    p_sel = jnp.where(choose_even, ia, ib)
    idx = jnp.where(choose_even, 2 * ia, 2 * ib + 1)

    g_iota = lax.broadcasted_iota(jnp.int32, (BB, BB * P), 1)
    b_iota = lax.broadcasted_iota(jnp.int32, (BB, BB * P), 0)
    onehot = (g_iota == p_sel + b_iota * P).astype(jnp.float32)
    pair = jnp.dot(onehot, x.reshape(BB * P, 128),
                   preferred_element_type=jnp.float32)
    lo = pair[:, :D]
    hi = pair[:, D:]
    masked_ref[...] = jnp.where(choose_even, lo, hi)
    idx_ref[...] = idx


@jax.jit
def kernel(x):
    x2 = x.reshape(B, P, 128)
    grid = (B // BB,)
    masked, idx = pl.pallas_call(
        _body,
        grid=grid,
        in_specs=[pl.BlockSpec((BB, P, 128), lambda i: (i, 0, 0))],
        out_specs=[
            pl.BlockSpec((BB, D), lambda i: (i, 0)),
            pl.BlockSpec((BB, 1), lambda i: (i, 0)),
        ],
        out_shape=[
            jax.ShapeDtypeStruct((B, D), jnp.float32),
            jax.ShapeDtypeStruct((B, 1), jnp.int32),
        ],
    )(x2)
    return masked[:, None, :], idx.reshape(B)


# speedup vs baseline: 1.1751x; 1.1751x over previous
import jax
import jax.numpy as jnp
from jax import lax
from jax.experimental import pallas as pl

B, C, D = 1024, 512, 64
BB = 32


def _body(x_ref, masked_ref, idx_ref):
    x = x_ref[...]
    s = jnp.sum(x * x, axis=2)
    smax = jnp.max(s, axis=1, keepdims=True)
    c_iota = lax.broadcasted_iota(jnp.int32, (BB, C), 1)
    idx = jnp.min(jnp.where(s >= smax, c_iota, C), axis=1, keepdims=True)
    g_iota = lax.broadcasted_iota(jnp.int32, (BB, BB * C), 1)
    b_iota = lax.broadcasted_iota(jnp.int32, (BB, BB * C), 0)
    onehot = (g_iota == idx + b_iota * C).astype(jnp.float32)
    masked = jnp.dot(onehot, x.reshape(BB * C, D),
                     preferred_element_type=jnp.float32)
    masked_ref[...] = masked
    idx_ref[...] = idx


@jax.jit
def kernel(x):
    grid = (B // BB,)
    masked, idx = pl.pallas_call(
        _body,
        grid=grid,
        in_specs=[pl.BlockSpec((BB, C, D), lambda i: (i, 0, 0))],
        out_specs=[
            pl.BlockSpec((BB, D), lambda i: (i, 0)),
            pl.BlockSpec((BB, 1), lambda i: (i, 0)),
        ],
        out_shape=[
            jax.ShapeDtypeStruct((B, D), jnp.float32),
            jax.ShapeDtypeStruct((B, 1), jnp.int32),
        ],
    )(x)
    return masked[:, None, :], idx.reshape(B)


# speedup vs baseline: 6.6454x; 5.6552x over previous
import jax
import jax.numpy as jnp
from jax import lax
from jax.experimental import pallas as pl

B, C, D = 1024, 512, 64
BB = 32
G = B // BB


def _body(xt_ref, masked_ref, idx_ref):
    xt = xt_ref[...]
    s = jnp.sum(xt * xt, axis=1)
    smax = jnp.max(s, axis=1, keepdims=True)
    c_iota = lax.broadcasted_iota(jnp.int32, (BB, C), 1)
    idx = jnp.min(jnp.where(s >= smax, c_iota, C), axis=1, keepdims=True)
    oh = (c_iota == idx).astype(jnp.float32)
    r = lax.dot_general(oh, xt.reshape(BB * D, C),
                        dimension_numbers=(((1,), (1,)), ((), ())),
                        preferred_element_type=jnp.float32)
    b_col = lax.broadcasted_iota(jnp.int32, (BB, 1), 0)
    acc = jnp.zeros((BB, D), jnp.float32)
    for j in range(BB):
        acc = acc + jnp.where(b_col == j, r[:, j * D:(j + 1) * D], 0.0)
    masked_ref[...] = acc
    idx_ref[...] = idx


@jax.jit
def kernel(x):
    xt = jnp.transpose(x, (0, 2, 1))
    masked, idx = pl.pallas_call(
        _body,
        grid=(G,),
        in_specs=[pl.BlockSpec((BB, D, C), lambda i: (i, 0, 0))],
        out_specs=[
            pl.BlockSpec((BB, D), lambda i: (i, 0)),
            pl.BlockSpec((BB, 1), lambda i: (i, 0)),
        ],
        out_shape=[
            jax.ShapeDtypeStruct((B, D), jnp.float32),
            jax.ShapeDtypeStruct((B, 1), jnp.int32),
        ],
    )(xt)
    return masked[:, None, :], idx.reshape(B)
